# baseline (speedup 1.0000x reference)
import functools

import jax
import jax.numpy as jnp
from jax import lax
from jax.experimental import pallas as pl
from jax.experimental.pallas import tpu as pltpu
from jax.experimental.pallas import tpu_sc as plsc

N = 10000
E = 320000
D = 128

NC = 2
NS = 16
L = 16
NW = NC * NS

NB = 10240
CHUNK = 64
NCHUNK = 160
EPT = CHUNK * NCHUNK
EPAD = EPT * NW
RPT = NB // NS

RB = 400
GRID = N // RB

_mesh = plsc.VectorSubcoreMesh(core_axis_name="c", subcore_axis_name="s")


def _hist_body(dst_hbm, h0_hbm, h1_hbm, dstbuf, hist, shared, tmp, acc):
    c = lax.axis_index("c")
    s = lax.axis_index("s")
    wid = c * NS + s
    zeros16 = jnp.zeros((L,), jnp.float32)
    ones16 = jnp.ones((L,), jnp.float32)

    def zero_hist(i, carry):
        hist[pl.ds(pl.multiple_of(i * L, L), L)] = zeros16
        return carry

    lax.fori_loop(0, NB // L, zero_hist, 0)

    pltpu.sync_copy(dst_hbm.at[wid], dstbuf)

    def update(i, carry):
        for j in range(CHUNK // L):
            idx = dstbuf[i, pl.ds(j * L, L)]
            plsc.addupdate_scatter(hist, [idx], ones16)
        return carry

    lax.fori_loop(0, NCHUNK, update, 0)

    pltpu.sync_copy(hist, shared.at[s])
    plsc.subcore_barrier()

    rbase = pl.multiple_of(s * RPT, RPT)

    def zero_acc(i, carry):
        acc[pl.ds(pl.multiple_of(i * L, L), L)] = zeros16
        return carry

    lax.fori_loop(0, RPT // L, zero_acc, 0)

    for t in range(NS):
        pltpu.sync_copy(shared.at[t, pl.ds(rbase, RPT)], tmp)

        def add_tmp(i, carry):
            sl = pl.ds(pl.multiple_of(i * L, L), L)
            acc[sl] = acc[sl] + tmp[sl]
            return carry

        lax.fori_loop(0, RPT // L, add_tmp, 0)

    @pl.when(c == 0)
    def _():
        pltpu.sync_copy(acc, h0_hbm.at[pl.ds(rbase, RPT)])

    @pl.when(c == 1)
    def _():
        pltpu.sync_copy(acc, h1_hbm.at[pl.ds(rbase, RPT)])


_hist = pl.kernel(
    _hist_body,
    out_type=[
        jax.ShapeDtypeStruct((NB,), jnp.float32),
        jax.ShapeDtypeStruct((NB,), jnp.float32),
    ],
    mesh=_mesh,
    scratch_types=[
        pltpu.VMEM((NCHUNK, CHUNK), jnp.int32),
        pltpu.VMEM((NB,), jnp.float32),
        pltpu.VMEM_SHARED((NS, NB), jnp.float32),
        pltpu.VMEM((RPT,), jnp.float32),
        pltpu.VMEM((RPT,), jnp.float32),
    ],
    compiler_params=pltpu.CompilerParams(needs_layout_passes=False),
)


NBUF = 4
SEGCH = 16
CA = 304
CB = 16
NSEG_A = CA // SEGCH
NSEG_B = CB // SEGCH
EPT_A = CA * CHUNK
EPT_B = CB * CHUNK
assert NS * (EPT_A + EPT_B) == EPAD


def _agg_body(h2_hbm, srcA_hbm, dstA_hbm, srcB_hbm, dstB_hbm,
              p0_hbm, p1_hbm,
              srcbuf, dstbuf, rb0, rb1, rb2, rb3, acc,
              gs0, gs1, gs2, gs3, ss0, ss1, ss2, ss3):
    c = lax.axis_index("c")
    s = lax.axis_index("s")
    rowbufs = [rb0, rb1, rb2, rb3]
    gsems = [gs0, gs1, gs2, gs3]
    ssems = [ss0, ss1, ss2, ss3]
    zeros16 = jnp.zeros((L,), jnp.float32)

    def zero_rb0(i, carry):
        for j in range(D // L):
            rb0[i, pl.ds(j * L, L)] = zeros16
        return carry

    rbase = pl.multiple_of(s * RPT, RPT)

    lax.fori_loop(0, CHUNK, zero_rb0, 0)

    def zero_acc(i, carry):
        off = pl.multiple_of(rbase + i * CHUNK, CHUNK)
        pltpu.sync_copy(rb0, acc.at[pl.ds(off, CHUNK)])
        return carry

    lax.fori_loop(0, RPT // CHUNK, zero_acc, 0)

    plsc.subcore_barrier()

    def pipeline(src_hbm, dst_hbm, nseg):
        for seg in range(nseg):
            pltpu.sync_copy(src_hbm.at[s, seg], srcbuf)
            pltpu.sync_copy(dst_hbm.at[s, seg], dstbuf)
            for k in range(NBUF):
                pltpu.async_copy(h2_hbm.at[srcbuf.at[k]], rowbufs[k],
                                 gsems[k])

            def group_body(g, carry):
                for k in range(NBUF):
                    j = g * NBUF + k
                    pltpu.make_async_copy(
                        h2_hbm.at[srcbuf.at[j]], rowbufs[k], gsems[k]).wait()
                    pltpu.async_copy(
                        rowbufs[k], acc.at[dstbuf.at[j]], ssems[k], add=True)
                for k in range(NBUF):
                    j = g * NBUF + k
                    pltpu.make_async_copy(
                        rowbufs[k], acc.at[dstbuf.at[j]], ssems[k]).wait()
                    nj = j + NBUF

                    @pl.when(nj < SEGCH)
                    def _():
                        pltpu.async_copy(
                            h2_hbm.at[srcbuf.at[nj]], rowbufs[k], gsems[k])
                return carry

            lax.fori_loop(0, SEGCH // NBUF, group_body, 0)

    @pl.when(c == 0)
    def _():
        pipeline(srcA_hbm, dstA_hbm, NSEG_A)

    @pl.when(c == 1)
    def _():
        pipeline(srcB_hbm, dstB_hbm, NSEG_B)

    plsc.subcore_barrier()

    @pl.when(c == 0)
    def _():
        pltpu.sync_copy(acc.at[pl.ds(rbase, RPT)], p0_hbm.at[pl.ds(rbase, RPT)])

    @pl.when(c == 1)
    def _():
        pltpu.sync_copy(acc.at[pl.ds(rbase, RPT)], p1_hbm.at[pl.ds(rbase, RPT)])


_agg = pl.kernel(
    _agg_body,
    out_type=[
        jax.ShapeDtypeStruct((NB, D), jnp.float32),
        jax.ShapeDtypeStruct((NB, D), jnp.float32),
    ],
    mesh=_mesh,
    scratch_types=(
        [
            pltpu.VMEM((SEGCH, CHUNK), jnp.int32),
            pltpu.VMEM((SEGCH, CHUNK), jnp.int32),
        ]
        + [pltpu.VMEM((CHUNK, D), jnp.float32)] * NBUF
        + [pltpu.VMEM_SHARED((NB, D), jnp.float32)]
        + [pltpu.SemaphoreType.DMA] * (2 * NBUF)
    ),
)


def _matmul_body(x_ref, w_ref, h_ref):
    h_ref[...] = jnp.dot(x_ref[...], w_ref[...],
                         preferred_element_type=jnp.float32,
                         precision=lax.Precision.HIGHEST)


_matmul = pl.pallas_call(
    _matmul_body,
    grid=(GRID,),
    in_specs=[
        pl.BlockSpec((RB, D), lambda i: (i, 0)),
        pl.BlockSpec((D, D), lambda i: (0, 0)),
    ],
    out_specs=pl.BlockSpec((RB, D), lambda i: (i, 0)),
    out_shape=jax.ShapeDtypeStruct((N, D), jnp.float32),
)


def _scale_body(h_ref, b_ref, h0_ref, h1_ref, h2_ref, base_ref, s_ref):
    h = h_ref[...]
    deg = h0_ref[...] + h1_ref[...] + 1.0
    sc = lax.rsqrt(deg)
    h2_ref[...] = h * sc
    base_ref[...] = h * (sc * sc) + b_ref[...]
    s_ref[...] = sc


_scale = pl.pallas_call(
    _scale_body,
    grid=(GRID,),
    in_specs=[
        pl.BlockSpec((RB, D), lambda i: (i, 0)),
        pl.BlockSpec((1, D), lambda i: (0, 0)),
        pl.BlockSpec((RB, 1), lambda i: (i, 0)),
        pl.BlockSpec((RB, 1), lambda i: (i, 0)),
    ],
    out_specs=[
        pl.BlockSpec((RB, D), lambda i: (i, 0)),
        pl.BlockSpec((RB, D), lambda i: (i, 0)),
        pl.BlockSpec((RB, 1), lambda i: (i, 0)),
    ],
    out_shape=[
        jax.ShapeDtypeStruct((N, D), jnp.float32),
        jax.ShapeDtypeStruct((N, D), jnp.float32),
        jax.ShapeDtypeStruct((N, 1), jnp.float32),
    ],
)


def _combine_body(base_ref, p0_ref, p1_ref, s_ref, o_ref):
    o_ref[...] = base_ref[...] + (p0_ref[...] + p1_ref[...]) * s_ref[...]


_combine = pl.pallas_call(
    _combine_body,
    grid=(GRID,),
    in_specs=[
        pl.BlockSpec((RB, D), lambda i: (i, 0)),
        pl.BlockSpec((RB, D), lambda i: (i, 0)),
        pl.BlockSpec((RB, D), lambda i: (i, 0)),
        pl.BlockSpec((RB, 1), lambda i: (i, 0)),
    ],
    out_specs=pl.BlockSpec((RB, D), lambda i: (i, 0)),
    out_shape=jax.ShapeDtypeStruct((N, D), jnp.float32),
)


def kernel(x, edge_index, W, b):
    src = edge_index[0]
    dst = edge_index[1]
    pad = EPAD - E
    srcp = jnp.concatenate([src, jnp.zeros((pad,), jnp.int32)])
    dstp = jnp.concatenate([dst, jnp.full((pad,), N, jnp.int32)])

    h = _matmul(x, W)
    h0, h1 = _hist(dstp.reshape(NW, NCHUNK, CHUNK))
    h2, base, s2d = _scale(h, b.reshape(1, D),
                           h0.reshape(NB, 1), h1.reshape(NB, 1))
    ea = NS * EPT_A
    p0, p1 = _agg(h2,
                  srcp[:ea].reshape(NS, NSEG_A, SEGCH, CHUNK),
                  dstp[:ea].reshape(NS, NSEG_A, SEGCH, CHUNK),
                  srcp[ea:].reshape(NS, NSEG_B, SEGCH, CHUNK),
                  dstp[ea:].reshape(NS, NSEG_B, SEGCH, CHUNK))
    return _combine(base, p0, p1, s2d)


# speedup vs baseline: 1.0168x; 1.0168x over previous
import functools

import jax
import jax.numpy as jnp
from jax import lax
from jax.experimental import pallas as pl
from jax.experimental.pallas import tpu as pltpu
from jax.experimental.pallas import tpu_sc as plsc

N = 10000
E = 320000
D = 128

NC = 2
NS = 16
L = 16
NW = NC * NS

NB = 10240
CHUNK = 64
NCHUNK = 160
EPT = CHUNK * NCHUNK
EPAD = EPT * NW
RPT = NB // NS

RB = 400
GRID = N // RB

_mesh = plsc.VectorSubcoreMesh(core_axis_name="c", subcore_axis_name="s")


def _hist_body(dst_hbm, h0_hbm, h1_hbm, dstbuf, hist, shared, tmp, acc):
    c = lax.axis_index("c")
    s = lax.axis_index("s")
    wid = c * NS + s
    zeros16 = jnp.zeros((L,), jnp.float32)
    ones16 = jnp.ones((L,), jnp.float32)

    def zero_hist(i, carry):
        hist[pl.ds(pl.multiple_of(i * L, L), L)] = zeros16
        return carry

    lax.fori_loop(0, NB // L, zero_hist, 0)

    pltpu.sync_copy(dst_hbm.at[wid], dstbuf)

    def update(i, carry):
        for j in range(CHUNK // L):
            idx = dstbuf[i, pl.ds(j * L, L)]
            plsc.addupdate_scatter(hist, [idx], ones16)
        return carry

    lax.fori_loop(0, NCHUNK, update, 0)

    pltpu.sync_copy(hist, shared.at[s])
    plsc.subcore_barrier()

    rbase = pl.multiple_of(s * RPT, RPT)

    def zero_acc(i, carry):
        acc[pl.ds(pl.multiple_of(i * L, L), L)] = zeros16
        return carry

    lax.fori_loop(0, RPT // L, zero_acc, 0)

    for t in range(NS):
        pltpu.sync_copy(shared.at[t, pl.ds(rbase, RPT)], tmp)

        def add_tmp(i, carry):
            sl = pl.ds(pl.multiple_of(i * L, L), L)
            acc[sl] = acc[sl] + tmp[sl]
            return carry

        lax.fori_loop(0, RPT // L, add_tmp, 0)

    @pl.when(c == 0)
    def _():
        pltpu.sync_copy(acc, h0_hbm.at[pl.ds(rbase, RPT)])

    @pl.when(c == 1)
    def _():
        pltpu.sync_copy(acc, h1_hbm.at[pl.ds(rbase, RPT)])


_hist = pl.kernel(
    _hist_body,
    out_type=[
        jax.ShapeDtypeStruct((NB,), jnp.float32),
        jax.ShapeDtypeStruct((NB,), jnp.float32),
    ],
    mesh=_mesh,
    scratch_types=[
        pltpu.VMEM((NCHUNK, CHUNK), jnp.int32),
        pltpu.VMEM((NB,), jnp.float32),
        pltpu.VMEM_SHARED((NS, NB), jnp.float32),
        pltpu.VMEM((RPT,), jnp.float32),
        pltpu.VMEM((RPT,), jnp.float32),
    ],
    compiler_params=pltpu.CompilerParams(needs_layout_passes=False),
)


NBUF = 4
SEGCH = 16
CA = 304
CB = 16
NSEG_A = CA // SEGCH
NSEG_B = CB // SEGCH
EPT_A = CA * CHUNK
EPT_B = CB * CHUNK
assert NS * (EPT_A + EPT_B) == EPAD


def _agg_body(h2_hbm, srcA_hbm, dstA_hbm, srcB_hbm, dstB_hbm,
              p0_hbm, p1_hbm,
              srcbuf, dstbuf, rb0, rb1, rb2, rb3, acc,
              gs0, gs1, gs2, gs3, ss0, ss1, ss2, ss3):
    c = lax.axis_index("c")
    s = lax.axis_index("s")
    rowbufs = [rb0, rb1, rb2, rb3]
    gsems = [gs0, gs1, gs2, gs3]
    ssems = [ss0, ss1, ss2, ss3]
    zeros16 = jnp.zeros((L,), jnp.float32)

    def zero_rb0(i, carry):
        for j in range(D // L):
            rb0[i, pl.ds(j * L, L)] = zeros16
        return carry

    rbase = pl.multiple_of(s * RPT, RPT)

    lax.fori_loop(0, CHUNK, zero_rb0, 0)

    def zero_acc(i, carry):
        off = pl.multiple_of(rbase + i * CHUNK, CHUNK)
        pltpu.sync_copy(rb0, acc.at[pl.ds(off, CHUNK)])
        return carry

    lax.fori_loop(0, RPT // CHUNK, zero_acc, 0)

    plsc.subcore_barrier()

    def pipeline(src_hbm, dst_hbm, nseg):
        for seg in range(nseg):
            pltpu.sync_copy(src_hbm.at[s, seg], srcbuf)
            pltpu.sync_copy(dst_hbm.at[s, seg], dstbuf)
            for k in range(NBUF):
                pltpu.async_copy(h2_hbm.at[srcbuf.at[k]], rowbufs[k],
                                 gsems[k])

            def group_body(g, carry):
                for k in range(NBUF):
                    j = g * NBUF + k
                    pltpu.make_async_copy(
                        h2_hbm.at[srcbuf.at[j]], rowbufs[k], gsems[k]).wait()
                    pltpu.async_copy(
                        rowbufs[k], acc.at[dstbuf.at[j]], ssems[k], add=True)
                for k in range(NBUF):
                    j = g * NBUF + k
                    pltpu.make_async_copy(
                        rowbufs[k], acc.at[dstbuf.at[j]], ssems[k]).wait()
                    nj = j + NBUF

                    @pl.when(nj < SEGCH)
                    def _():
                        pltpu.async_copy(
                            h2_hbm.at[srcbuf.at[nj]], rowbufs[k], gsems[k])
                return carry

            lax.fori_loop(0, SEGCH // NBUF, group_body, 0)

    @pl.when(c == 0)
    def _():
        pipeline(srcA_hbm, dstA_hbm, NSEG_A)

    @pl.when(c == 1)
    def _():
        pipeline(srcB_hbm, dstB_hbm, NSEG_B)

    plsc.subcore_barrier()

    @pl.when(c == 0)
    def _():
        pltpu.sync_copy(acc.at[pl.ds(rbase, RPT)], p0_hbm.at[pl.ds(rbase, RPT)])

    @pl.when(c == 1)
    def _():
        pltpu.sync_copy(acc.at[pl.ds(rbase, RPT)], p1_hbm.at[pl.ds(rbase, RPT)])


_agg = pl.kernel(
    _agg_body,
    out_type=[
        jax.ShapeDtypeStruct((NB, D), jnp.float32),
        jax.ShapeDtypeStruct((NB, D), jnp.float32),
    ],
    mesh=_mesh,
    scratch_types=(
        [
            pltpu.VMEM((SEGCH, CHUNK), jnp.int32),
            pltpu.VMEM((SEGCH, CHUNK), jnp.int32),
        ]
        + [pltpu.VMEM((CHUNK, D), jnp.float32)] * NBUF
        + [pltpu.VMEM_SHARED((NB, D), jnp.float32)]
        + [pltpu.SemaphoreType.DMA] * (2 * NBUF)
    ),
)


def _transform_body(x_ref, w_ref, b_ref, h0_ref, h1_ref,
                    h2_ref, base_ref, s_ref):
    h = jnp.dot(x_ref[...], w_ref[...],
                preferred_element_type=jnp.float32,
                precision=lax.Precision.HIGHEST)
    deg = h0_ref[...] + h1_ref[...] + 1.0
    sc = lax.rsqrt(deg)
    h2_ref[...] = h * sc
    base_ref[...] = h * (sc * sc) + b_ref[...]
    s_ref[...] = sc


_transform = pl.pallas_call(
    _transform_body,
    grid=(GRID,),
    in_specs=[
        pl.BlockSpec((RB, D), lambda i: (i, 0)),
        pl.BlockSpec((D, D), lambda i: (0, 0)),
        pl.BlockSpec((1, D), lambda i: (0, 0)),
        pl.BlockSpec((RB, 1), lambda i: (i, 0)),
        pl.BlockSpec((RB, 1), lambda i: (i, 0)),
    ],
    out_specs=[
        pl.BlockSpec((RB, D), lambda i: (i, 0)),
        pl.BlockSpec((RB, D), lambda i: (i, 0)),
        pl.BlockSpec((RB, 1), lambda i: (i, 0)),
    ],
    out_shape=[
        jax.ShapeDtypeStruct((N, D), jnp.float32),
        jax.ShapeDtypeStruct((N, D), jnp.float32),
        jax.ShapeDtypeStruct((N, 1), jnp.float32),
    ],
)


def _combine_body(base_ref, p0_ref, p1_ref, s_ref, o_ref):
    o_ref[...] = base_ref[...] + (p0_ref[...] + p1_ref[...]) * s_ref[...]


_combine = pl.pallas_call(
    _combine_body,
    grid=(GRID,),
    in_specs=[
        pl.BlockSpec((RB, D), lambda i: (i, 0)),
        pl.BlockSpec((RB, D), lambda i: (i, 0)),
        pl.BlockSpec((RB, D), lambda i: (i, 0)),
        pl.BlockSpec((RB, 1), lambda i: (i, 0)),
    ],
    out_specs=pl.BlockSpec((RB, D), lambda i: (i, 0)),
    out_shape=jax.ShapeDtypeStruct((N, D), jnp.float32),
)


def kernel(x, edge_index, W, b):
    src = edge_index[0]
    dst = edge_index[1]
    pad = EPAD - E
    srcp = jnp.concatenate([src, jnp.zeros((pad,), jnp.int32)])
    dstp = jnp.concatenate([dst, jnp.full((pad,), N, jnp.int32)])

    h0, h1 = _hist(dstp.reshape(NW, NCHUNK, CHUNK))
    h2, base, s2d = _transform(x, W, b.reshape(1, D),
                               h0.reshape(NB, 1), h1.reshape(NB, 1))
    ea = NS * EPT_A
    p0, p1 = _agg(h2,
                  srcp[:ea].reshape(NS, NSEG_A, SEGCH, CHUNK),
                  dstp[:ea].reshape(NS, NSEG_A, SEGCH, CHUNK),
                  srcp[ea:].reshape(NS, NSEG_B, SEGCH, CHUNK),
                  dstp[ea:].reshape(NS, NSEG_B, SEGCH, CHUNK))
    return _combine(base, p0, p1, s2d)
